# baseline (speedup 1.0000x reference)
import jax
import jax.numpy as jnp
from jax.experimental import pallas as pl
from jax.experimental.pallas import tpu as pltpu

N = 4096
B = 4
IN_F = 256
OUT_F = 256

BK = 512
MC = 1024


def _gcn_kernel(adj_ref, x_ref, w_ref, out_ref, acc_ref, x2d_ref, sem):
    k = pl.program_id(0)
    nk = pl.num_programs(0)

    for b in range(B):
        x2d_ref[:, b * IN_F : (b + 1) * IN_F] = x_ref[:, b, :].astype(
            jnp.bfloat16
        )

    xbf = x2d_ref[...]

    @pl.when(k == 0)
    def _first():
        for mc in range(N // MC):
            sl = slice(mc * MC, (mc + 1) * MC)
            acc_ref[sl, :] = jnp.dot(
                adj_ref[sl, :].astype(jnp.bfloat16),
                xbf,
                preferred_element_type=jnp.float32,
            )

    @pl.when(k > 0)
    def _accum():
        for mc in range(N // MC):
            sl = slice(mc * MC, (mc + 1) * MC)
            acc_ref[sl, :] += jnp.dot(
                adj_ref[sl, :].astype(jnp.bfloat16),
                xbf,
                preferred_element_type=jnp.float32,
            )

    @pl.when(k == nk - 1)
    def _finish():
        w = w_ref[...].astype(jnp.bfloat16)
        for b in range(B):
            cols = slice(b * IN_F, (b + 1) * IN_F)
            acc_ref[:, cols] = jnp.dot(
                acc_ref[:, cols].astype(jnp.bfloat16),
                w,
                preferred_element_type=jnp.float32,
            )
        out_copies = [
            pltpu.make_async_copy(
                acc_ref.at[:, b * OUT_F : (b + 1) * OUT_F],
                out_ref.at[:, b, :],
                sem,
            )
            for b in range(B)
        ]
        for c in out_copies:
            c.start()
        for c in out_copies:
            c.wait()


@jax.jit
def kernel(input, adj, weight):
    grid = (N // BK,)
    return pl.pallas_call(
        _gcn_kernel,
        grid=grid,
        in_specs=[
            pl.BlockSpec((N, BK), lambda k: (0, k)),
            pl.BlockSpec((BK, B, IN_F), lambda k: (k, 0, 0)),
            pl.BlockSpec((IN_F, OUT_F), lambda k: (0, 0)),
        ],
        out_specs=pl.BlockSpec((N, B, OUT_F), lambda k: (0, 0, 0)),
        out_shape=jax.ShapeDtypeStruct((N, B, OUT_F), jnp.float32),
        scratch_shapes=[
            pltpu.VMEM((N, B * IN_F), jnp.float32),
            pltpu.VMEM((BK, B * IN_F), jnp.bfloat16),
            pltpu.SemaphoreType.DMA,
        ],
    )(adj, input, weight)


# speedup vs baseline: 1.2736x; 1.2736x over previous
import jax
import jax.numpy as jnp
from jax.experimental import pallas as pl
from jax.experimental.pallas import tpu as pltpu

N = 4096
B = 4
IN_F = 256
OUT_F = 256

BK = 512
MC = 1024


def _gcn_kernel(adj_ref, x_ref, w_ref, out_ref, acc_ref, xbf_ref):
    k = pl.program_id(0)
    nk = pl.num_programs(0)

    for b in range(B):
        xbf_ref[:, b * IN_F : (b + 1) * IN_F] = x_ref[:, b, :].astype(
            jnp.bfloat16
        )

    @pl.when(k == 0)
    def _first():
        for mc in range(N // MC):
            sl = slice(mc * MC, (mc + 1) * MC)
            acc_ref[sl, :] = jnp.dot(
                adj_ref[sl, :].astype(jnp.bfloat16),
                xbf_ref[...],
                preferred_element_type=jnp.float32,
            )

    @pl.when(k > 0)
    def _accum():
        for mc in range(N // MC):
            sl = slice(mc * MC, (mc + 1) * MC)
            acc_ref[sl, :] += jnp.dot(
                adj_ref[sl, :].astype(jnp.bfloat16),
                xbf_ref[...],
                preferred_element_type=jnp.float32,
            )

    @pl.when(k == nk - 1)
    def _finish():
        w = w_ref[...].astype(jnp.bfloat16)
        for b in range(B):
            out_ref[:, b, :] = jnp.dot(
                acc_ref[:, b * IN_F : (b + 1) * IN_F].astype(jnp.bfloat16),
                w,
                preferred_element_type=jnp.float32,
            )


@jax.jit
def kernel(input, adj, weight):
    grid = (N // BK,)
    return pl.pallas_call(
        _gcn_kernel,
        grid=grid,
        in_specs=[
            pl.BlockSpec((N, BK), lambda k: (0, k)),
            pl.BlockSpec((BK, B, IN_F), lambda k: (k, 0, 0)),
            pl.BlockSpec((IN_F, OUT_F), lambda k: (0, 0)),
        ],
        out_specs=pl.BlockSpec((N, B, OUT_F), lambda k: (0, 0, 0)),
        out_shape=jax.ShapeDtypeStruct((N, B, OUT_F), jnp.float32),
        scratch_shapes=[
            pltpu.VMEM((N, B * IN_F), jnp.float32),
            pltpu.VMEM((BK, B * IN_F), jnp.bfloat16),
        ],
    )(adj, input, weight)
